# baseline (speedup 1.0000x reference)
import jax
import jax.numpy as jnp
from jax import lax
from jax.experimental import pallas as pl
from jax.experimental.pallas import tpu as pltpu
from jax.experimental.pallas import tpu_sc as plsc

B = 1024
SENT = 20
REG = 5
T = SENT * REG
TPAD = 112
CO = 1000
CA = 896
TW = 128
TOFF = CA
NPOOL = 10
OUTW = NPOOL * CO

NC = 2
NS = 16
NW = NC * NS
BPW = B // NW

LANES = 16
NCHUNK_A = CA // LANES
NCHUNK = 63

SLOT_OFF = (0, 40, 80)
SLOT_LEN = (40, 40, 24)
SLOT_SENT = (8, 8, 4)


def _sc_body(xp_hbm, table_hbm, tail_hbm, bias_hbm, out_hbm,
             idx_v, bias_v, outb_v, rows_a, rows_b, asems, bsems):
    wid = lax.axis_index("s") * NC + lax.axis_index("c")
    base_b = wid * BPW

    pltpu.sync_copy(xp_hbm.at[pl.ds(base_b, BPW)], idx_v)
    pltpu.sync_copy(bias_hbm, bias_v)

    iota = lax.iota(jnp.int32, LANES)

    def _split(j):
        n = SLOT_LEN[j]
        h = 24 if n == 40 else 16
        return (0, h), (h, n - h)

    def fire(bb, j):
        for (o, n) in _split(j):
            idx = idx_v.at[bb, pl.ds(SLOT_OFF[j] + o, n)]
            pltpu.async_copy(table_hbm.at[idx, pl.ds(0, CA)],
                             rows_a[j].at[pl.ds(o, n)], asems[j])
        idx = idx_v.at[bb, pl.ds(SLOT_OFF[j], SLOT_LEN[j])]
        pltpu.async_copy(tail_hbm.at[idx], rows_b[j], bsems[j])

    def wait(bb, j):
        for (o, n) in _split(j):
            idx = idx_v.at[bb, pl.ds(SLOT_OFF[j] + o, n)]
            pltpu.make_async_copy(table_hbm.at[idx, pl.ds(0, CA)],
                                  rows_a[j].at[pl.ds(o, n)], asems[j]).wait()
        idx = idx_v.at[bb, pl.ds(SLOT_OFF[j], SLOT_LEN[j])]
        pltpu.make_async_copy(tail_hbm.at[idx], rows_b[j], bsems[j]).wait()

    for j in range(3):
        fire(jnp.int32(0), j)

    zero16 = jnp.zeros((LANES,), jnp.float32)
    tailw = jnp.minimum(jnp.maximum(iota.astype(jnp.float32) - 7.0, 0.0), 1.0)

    @pl.loop(0, BPW)
    def _batch(bb):
        ssacc = zero16
        for j in range(3):
            wait(bb, j)
            ra, rb = rows_a[j], rows_b[j]

            wins = []
            for w in range(3 if j < 2 else 2):
                wins.append(idx_v[bb, pl.ds(SLOT_OFF[j] + w * LANES, LANES)])
            for k in range(SLOT_SENT[j] * REG):
                tok = wins[k // LANES][k % LANES]

                @pl.when(tok == 0)
                def _():
                    @pl.loop(0, NCHUNK_A)
                    def _za(c):
                        ra[k, pl.ds(c * LANES, LANES)] = zero16

                    @pl.loop(0, 7)
                    def _zb(c):
                        rb[k, pl.ds(c * LANES, LANES)] = zero16

            pool0 = SLOT_OFF[j] // 10

            def chunk_body(coff, buf, base, ss, wvec=None):
                loff = coff - base
                bvec = bias_v[pl.ds(coff, LANES)]
                obase = (coff + iota) * NPOOL
                for half in range(SLOT_SENT[j] // 2):
                    accs = []
                    for sj in (2 * half, 2 * half + 1):
                        acc = buf[sj * REG, pl.ds(loff, LANES)]
                        for r in range(1, REG):
                            acc = acc + buf[sj * REG + r, pl.ds(loff, LANES)]
                        accs.append(jnp.maximum(acc + bvec, 0.0))
                    pooled = (accs[0] + accs[1]) * 0.5
                    plsc.store_scatter(outb_v, [obase + pool0 + half], pooled)
                    sq = pooled * pooled
                    if wvec is not None:
                        sq = sq * wvec
                    ss = ss + sq
                return ss

            @plsc.parallel_loop(0, NCHUNK_A, unroll=4, carry=ssacc)
            def _chunks_a(c, ss):
                return chunk_body(c * LANES, ra, 0, ss)

            @plsc.parallel_loop(NCHUNK_A, NCHUNK - 1, unroll=2,
                                carry=_chunks_a)
            def _chunks_b(c, ss):
                return chunk_body(c * LANES, rb, TOFF, ss)

            ssacc = chunk_body(CO - LANES, rb, TOFF, _chunks_b, wvec=tailw)

            @pl.when(bb + 1 < BPW)
            def _():
                fire(bb + 1, j)

        total = jnp.sum(ssacc)
        s = jnp.full((LANES,), 1.0 + total, jnp.float32)
        yi = jnp.int32(0x5F3759DF) - (plsc.bitcast(s, jnp.int32) >> 1)
        y = plsc.bitcast(yi, jnp.float32)
        for _ in range(3):
            y = y * (1.5 - 0.5 * s * y * y)

        @plsc.parallel_loop(0, OUTW // LANES, unroll=4)
        def _scale(k):
            off = k * LANES
            outb_v[pl.ds(off, LANES)] = outb_v[pl.ds(off, LANES)] * y

        pltpu.sync_copy(outb_v, out_hbm.at[base_b + bb])


@jax.jit
def kernel(x, embed_weight, bias):
    xflat = x.reshape(B, T)
    xp = jnp.pad(xflat, ((0, 0), (0, TPAD - T)))
    tail_t = jnp.pad(embed_weight[:, TOFF:], ((0, 0), (0, TW - (CO - TOFF))))
    bias_flat = bias.reshape(CO)

    mesh = plsc.VectorSubcoreMesh(core_axis_name="c", subcore_axis_name="s")
    f = pl.kernel(
        _sc_body,
        out_type=jax.ShapeDtypeStruct((B, OUTW), jnp.float32),
        mesh=mesh,
        scratch_types=[
            pltpu.VMEM((BPW, TPAD), jnp.int32),
            pltpu.VMEM((CO,), jnp.float32),
            pltpu.VMEM((OUTW,), jnp.float32),
            [pltpu.VMEM((SLOT_LEN[j], CA), jnp.float32) for j in range(3)],
            [pltpu.VMEM((SLOT_LEN[j], TW), jnp.float32) for j in range(3)],
            [pltpu.SemaphoreType.DMA for _ in range(3)],
            [pltpu.SemaphoreType.DMA for _ in range(3)],
        ],
        compiler_params=pltpu.CompilerParams(needs_layout_passes=False),
    )
    return f(xp, embed_weight, tail_t, bias_flat)


# speedup vs baseline: 1.0193x; 1.0193x over previous
import jax
import jax.numpy as jnp
from jax import lax
from jax.experimental import pallas as pl
from jax.experimental.pallas import tpu as pltpu
from jax.experimental.pallas import tpu_sc as plsc

B = 1024
SENT = 20
REG = 5
T = SENT * REG
TPAD = 112
CO = 1000
CA = 896
TW = 128
TOFF = CO - TW
NPOOL = 10
OUTW = NPOOL * CO

NC = 2
NS = 16
NW = NC * NS
BPW = B // NW

LANES = 16
NCHUNK_A = CA // LANES
NCHUNK = 63

SLOT_OFF = (0, 40, 80)
SLOT_LEN = (40, 40, 24)
SLOT_SENT = (8, 8, 4)


def _sc_body(xp_hbm, table_hbm, tail_hbm, bias_hbm, out_hbm,
             idx_v, bias_v, outb_v, rows_a, rows_b, asems, bsems):
    wid = lax.axis_index("s") * NC + lax.axis_index("c")
    base_b = wid * BPW

    pltpu.sync_copy(xp_hbm.at[pl.ds(base_b, BPW)], idx_v)
    pltpu.sync_copy(bias_hbm, bias_v)

    iota = lax.iota(jnp.int32, LANES)

    def fire(bb, j):
        idx = idx_v.at[bb, pl.ds(SLOT_OFF[j], SLOT_LEN[j])]
        pltpu.async_copy(table_hbm.at[idx, pl.ds(0, CA)], rows_a[j], asems[j])
        pltpu.async_copy(tail_hbm.at[idx], rows_b[j], bsems[j])

    def wait(bb, j):
        idx = idx_v.at[bb, pl.ds(SLOT_OFF[j], SLOT_LEN[j])]
        pltpu.make_async_copy(table_hbm.at[idx, pl.ds(0, CA)], rows_a[j],
                              asems[j]).wait()
        pltpu.make_async_copy(tail_hbm.at[idx], rows_b[j], bsems[j]).wait()

    for j in range(3):
        fire(jnp.int32(0), j)

    zero16 = jnp.zeros((LANES,), jnp.float32)
    tailw = jnp.minimum(jnp.maximum(iota.astype(jnp.float32) - 7.0, 0.0), 1.0)

    @pl.loop(0, BPW)
    def _batch(bb):
        ssacc = zero16
        for j in range(3):
            wait(bb, j)
            ra, rb = rows_a[j], rows_b[j]

            wins = []
            for w in range(3 if j < 2 else 2):
                wins.append(idx_v[bb, pl.ds(SLOT_OFF[j] + w * LANES, LANES)])
            for k in range(SLOT_SENT[j] * REG):
                tok = wins[k // LANES][k % LANES]

                @pl.when(tok == 0)
                def _():
                    @pl.loop(0, NCHUNK_A)
                    def _za(c):
                        ra[k, pl.ds(c * LANES, LANES)] = zero16

                    @pl.loop(0, 7)
                    def _zb(c):
                        rb[k, pl.ds(c * LANES, LANES)] = zero16

            pool0 = SLOT_OFF[j] // 10

            def chunk_body(coff, buf, base, ss, wvec=None):
                loff = coff - base
                bvec = bias_v[pl.ds(coff, LANES)]
                obase = (coff + iota) * NPOOL
                for half in range(SLOT_SENT[j] // 2):
                    accs = []
                    for sj in (2 * half, 2 * half + 1):
                        acc = buf[sj * REG, pl.ds(loff, LANES)]
                        for r in range(1, REG):
                            acc = acc + buf[sj * REG + r, pl.ds(loff, LANES)]
                        accs.append(jnp.maximum(acc + bvec, 0.0))
                    pooled = (accs[0] + accs[1]) * 0.5
                    plsc.store_scatter(outb_v, [obase + pool0 + half], pooled)
                    sq = pooled * pooled
                    if wvec is not None:
                        sq = sq * wvec
                    ss = ss + sq
                return ss

            @plsc.parallel_loop(0, NCHUNK_A, unroll=4, carry=ssacc)
            def _chunks_a(c, ss):
                return chunk_body(c * LANES, ra, 0, ss)

            @plsc.parallel_loop(NCHUNK_A, NCHUNK - 1, unroll=2,
                                carry=_chunks_a)
            def _chunks_b(c, ss):
                return chunk_body(c * LANES, rb, TOFF, ss)

            ssacc = chunk_body(CO - LANES, rb, TOFF, _chunks_b, wvec=tailw)

            @pl.when(bb + 1 < BPW)
            def _():
                fire(bb + 1, j)

        total = jnp.sum(ssacc)
        s = jnp.full((LANES,), 1.0 + total, jnp.float32)
        yi = jnp.int32(0x5F3759DF) - (plsc.bitcast(s, jnp.int32) >> 1)
        y = plsc.bitcast(yi, jnp.float32)
        for _ in range(3):
            y = y * (1.5 - 0.5 * s * y * y)

        @plsc.parallel_loop(0, OUTW // LANES, unroll=4)
        def _scale(k):
            off = k * LANES
            outb_v[pl.ds(off, LANES)] = outb_v[pl.ds(off, LANES)] * y

        pltpu.sync_copy(outb_v, out_hbm.at[base_b + bb])


@jax.jit
def kernel(x, embed_weight, bias):
    xflat = x.reshape(B, T)
    xp = jnp.pad(xflat, ((0, 0), (0, TPAD - T)))
    tail_t = embed_weight[:, TOFF:]
    bias_flat = bias.reshape(CO)

    mesh = plsc.VectorSubcoreMesh(core_axis_name="c", subcore_axis_name="s")
    f = pl.kernel(
        _sc_body,
        out_type=jax.ShapeDtypeStruct((B, OUTW), jnp.float32),
        mesh=mesh,
        scratch_types=[
            pltpu.VMEM((BPW, TPAD), jnp.int32),
            pltpu.VMEM((CO,), jnp.float32),
            pltpu.VMEM((OUTW,), jnp.float32),
            [pltpu.VMEM((SLOT_LEN[j], CA), jnp.float32) for j in range(3)],
            [pltpu.VMEM((SLOT_LEN[j], TW), jnp.float32) for j in range(3)],
            [pltpu.SemaphoreType.DMA for _ in range(3)],
            [pltpu.SemaphoreType.DMA for _ in range(3)],
        ],
        compiler_params=pltpu.CompilerParams(needs_layout_passes=False),
    )
    return f(xp, embed_weight, tail_t, bias_flat)
